# baseline (speedup 1.0000x reference)
import functools

import jax
import jax.numpy as jnp
from jax import lax
from jax.experimental import pallas as pl
from jax.experimental.pallas import tpu as pltpu
from jax.experimental.pallas import tpu_sc as plsc

N = 10000
NP = 10240
E = 160000
D = 256
NTILES = 32
CPT = D // NTILES
BN = 512
CH = 1024
E_PAD = 163840
EPW = E_PAD // NTILES

_mesh = plsc.VectorSubcoreMesh(
    core_axis_name="c", subcore_axis_name="s", num_cores=2, num_subcores=16
)


def _wid():
    return lax.axis_index("s") * 2 + lax.axis_index("c")


@functools.partial(
    pl.kernel,
    out_type=jax.ShapeDtypeStruct((NTILES, NP), jnp.float32),
    mesh=_mesh,
    compiler_params=pltpu.CompilerParams(needs_layout_passes=False),
    scratch_types=[
        pltpu.VMEM((EPW,), jnp.int32),
        pltpu.VMEM((EPW,), jnp.float32),
        pltpu.VMEM((NP,), jnp.float32),
    ],
)
def _deg_kernel(col_hbm, ew_hbm, out_hbm, col_v, ew_v, acc_v):
    w = _wid()
    base = w * EPW
    pltpu.sync_copy(col_hbm.at[pl.ds(base, EPW)], col_v)
    pltpu.sync_copy(ew_hbm.at[pl.ds(base, EPW)], ew_v)

    @plsc.parallel_loop(0, NP // 16, unroll=4)
    def zero_body(i):
        acc_v[pl.ds(i * 16, 16)] = jnp.zeros((16,), jnp.float32)

    @plsc.parallel_loop(0, EPW // 16, unroll=4)
    def ebody(j):
        c16 = col_v[pl.ds(j * 16, 16)]
        w16 = ew_v[pl.ds(j * 16, 16)]
        plsc.addupdate_scatter(acc_v, [c16], w16)
    pltpu.sync_copy(acc_v, out_hbm.at[w])


NCHUNK = E_PAD // CH


@functools.partial(
    pl.kernel,
    out_type=jax.ShapeDtypeStruct((NP, D), jnp.float32),
    mesh=_mesh,
    compiler_params=pltpu.CompilerParams(
        needs_layout_passes=False, use_tc_tiling_on_sc=False
    ),
    scratch_types=[
        pltpu.VMEM((2, CH), jnp.int32),
        pltpu.VMEM((2, CH), jnp.int32),
        pltpu.VMEM((2, CH), jnp.float32),
        pltpu.VMEM((2, CH), jnp.int32),
        pltpu.VMEM((2, CH, CPT), jnp.float32),
        pltpu.VMEM((NP, CPT), jnp.float32),
        pltpu.SemaphoreType.DMA,
        pltpu.SemaphoreType.DMA,
        pltpu.SemaphoreType.DMA,
        pltpu.SemaphoreType.DMA,
    ],
)
def _edge_kernel(yf_hbm, row_hbm, col_hbm, ew_hbm, zero_hbm, out_hbm,
                 row_v, col_v, ew_v, idx_v, gat_v, acc_v, sr0, sr1, sg0, sg1):
    w = _wid()
    srs = (sr0, sr1)
    sgs = (sg0, sg1)
    iota16 = lax.iota(jnp.int32, 16)

    def rce_copies(ci, b):
        base = ci * CH
        return (
            pltpu.make_async_copy(row_hbm.at[pl.ds(base, CH)], row_v.at[b], srs[b]),
            pltpu.make_async_copy(col_hbm.at[pl.ds(base, CH)], col_v.at[b], srs[b]),
            pltpu.make_async_copy(ew_hbm.at[pl.ds(base, CH)], ew_v.at[b], srs[b]),
        )

    def start_rce(ci, b):
        for c in rce_copies(ci, b):
            c.start()

    def wait_rce(b):
        for c in rce_copies(0, b):
            c.wait()

    def make_idx(b):
        @plsc.parallel_loop(0, CH // 16, unroll=4)
        def ibody(j):
            r16 = row_v[b, pl.ds(j * 16, 16)]
            idx_v[b, pl.ds(j * 16, 16)] = r16 * NTILES + w

    def gat_copy(b):
        return pltpu.make_async_copy(yf_hbm.at[idx_v.at[b]], gat_v.at[b], sgs[b])

    def compute(b):
        @plsc.parallel_loop(0, CH // 16, unroll=8)
        def cbody(j):
            e16 = j * 16 + iota16
            c16 = col_v[b, pl.ds(j * 16, 16)]
            w16 = ew_v[b, pl.ds(j * 16, 16)]
            for c in range(CPT):
                cc = jnp.full((16,), c, jnp.int32)
                vals = plsc.load_gather(gat_v.at[b], [e16, cc]) * w16
                plsc.addupdate_scatter(acc_v, [c16, cc], vals)

    zc = pltpu.make_async_copy(zero_hbm, acc_v, sg0)
    zc.start()
    start_rce(0, 0)
    start_rce(1, 1)
    wait_rce(0)
    make_idx(0)
    zc.wait()
    gat_copy(0).start()

    def outer(g, _):
        for b in range(2):
            nb = 1 - b
            ci = g * 2 + b

            @pl.when(ci + 1 < NCHUNK)
            def _():
                wait_rce(nb)
                make_idx(nb)
                gat_copy(nb).start()

            gat_copy(b).wait()
            compute(b)

            @pl.when(ci + 2 < NCHUNK)
            def _():
                start_rce(ci + 2, b)

        return 0

    lax.fori_loop(0, NCHUNK // 2, outer, 0)
    pltpu.sync_copy(acc_v, out_hbm.at[:, pl.ds(w * CPT, CPT)])


def _mm_body(x_ref, w_ref, o_ref):
    o_ref[...] = jnp.dot(x_ref[...], w_ref[...], preferred_element_type=jnp.float32)


def _scale_body(p_ref, xl_ref, y_ref):
    deg = 1.0 + jnp.sum(p_ref[...], axis=0)
    dinv = jnp.where(deg > 0.0, lax.rsqrt(deg), 0.0)
    y_ref[...] = xl_ref[...] * dinv[:, None]


def _final_body(p_ref, s_ref, y_ref, b_ref, o_ref):
    deg = 1.0 + jnp.sum(p_ref[...], axis=0)
    dinv = jnp.where(deg > 0.0, lax.rsqrt(deg), 0.0)
    o_ref[...] = jnp.maximum(dinv[:, None] * (s_ref[...] + y_ref[...]) + b_ref[...], 0.0)


@jax.jit
def _impl(x, edge_index, edge_weight, W, b):
    row = edge_index[0].astype(jnp.int32)
    col = edge_index[1].astype(jnp.int32)
    pad = E_PAD - E
    rowp = jnp.concatenate([row, jnp.zeros((pad,), jnp.int32)])
    colp = jnp.concatenate([col, jnp.zeros((pad,), jnp.int32)])
    ewp = jnp.concatenate([edge_weight.astype(jnp.float32),
                           jnp.zeros((pad,), jnp.float32)])
    xp = jnp.concatenate([x.astype(jnp.float32),
                          jnp.zeros((NP - N, x.shape[1]), jnp.float32)])

    xlin = pl.pallas_call(
        _mm_body,
        grid=(NP // BN,),
        in_specs=[
            pl.BlockSpec((BN, D), lambda i: (i, 0)),
            pl.BlockSpec((D, D), lambda i: (0, 0)),
        ],
        out_specs=pl.BlockSpec((BN, D), lambda i: (i, 0)),
        out_shape=jax.ShapeDtypeStruct((NP, D), jnp.float32),
    )(xp, W)

    partials = _deg_kernel(colp, ewp)

    y = pl.pallas_call(
        _scale_body,
        grid=(NP // BN,),
        in_specs=[
            pl.BlockSpec((NTILES, BN), lambda i: (0, i)),
            pl.BlockSpec((BN, D), lambda i: (i, 0)),
        ],
        out_specs=pl.BlockSpec((BN, D), lambda i: (i, 0)),
        out_shape=jax.ShapeDtypeStruct((NP, D), jnp.float32),
    )(partials, xlin)

    yf = y.reshape(NP * NTILES, CPT)
    zero = jnp.zeros((NP, CPT), jnp.float32)
    s = _edge_kernel(yf, rowp, colp, ewp, zero)

    b2 = b.reshape(1, D).astype(jnp.float32)
    out = pl.pallas_call(
        _final_body,
        grid=(NP // BN,),
        in_specs=[
            pl.BlockSpec((NTILES, BN), lambda i: (0, i)),
            pl.BlockSpec((BN, D), lambda i: (i, 0)),
            pl.BlockSpec((BN, D), lambda i: (i, 0)),
            pl.BlockSpec((1, D), lambda i: (0, 0)),
        ],
        out_specs=pl.BlockSpec((BN, D), lambda i: (i, 0)),
        out_shape=jax.ShapeDtypeStruct((NP, D), jnp.float32),
    )(partials, s, y, b2)
    return out[:N]


def kernel(x, edge_index, edge_weight, W, b):
    return _impl(x, edge_index, edge_weight, W, b)


# speedup vs baseline: 1.4208x; 1.4208x over previous
import functools

import jax
import jax.numpy as jnp
from jax import lax
from jax.experimental import pallas as pl
from jax.experimental.pallas import tpu as pltpu
from jax.experimental.pallas import tpu_sc as plsc

N = 10000
NP = 10240
E = 160000
D = 256
NTILES = 32
CPT = D // NTILES
BN = 512
CH = 1024
E_PAD = 163840
EPW = E_PAD // NTILES

_mesh = plsc.VectorSubcoreMesh(
    core_axis_name="c", subcore_axis_name="s", num_cores=2, num_subcores=16
)

_DNUMS = lax.GatherDimensionNumbers(
    offset_dims=(), collapsed_slice_dims=(0,), start_index_map=(0,)
)


def _wid():
    return lax.axis_index("s") * 2 + lax.axis_index("c")


@functools.partial(
    pl.kernel,
    out_type=jax.ShapeDtypeStruct((NTILES, NP), jnp.float32),
    mesh=_mesh,
    compiler_params=pltpu.CompilerParams(needs_layout_passes=False),
    scratch_types=[
        pltpu.VMEM((EPW,), jnp.int32),
        pltpu.VMEM((EPW,), jnp.float32),
        pltpu.VMEM((NP,), jnp.float32),
    ],
)
def _deg_kernel(col_hbm, ew_hbm, out_hbm, col_v, ew_v, acc_v):
    w = _wid()
    base = w * EPW
    pltpu.sync_copy(col_hbm.at[pl.ds(base, EPW)], col_v)
    pltpu.sync_copy(ew_hbm.at[pl.ds(base, EPW)], ew_v)

    @plsc.parallel_loop(0, NP // 16, unroll=4)
    def zero_body(i):
        acc_v[pl.ds(i * 16, 16)] = jnp.zeros((16,), jnp.float32)

    @plsc.parallel_loop(0, EPW // 16, unroll=4)
    def ebody(j):
        c16 = col_v[pl.ds(j * 16, 16)]
        w16 = ew_v[pl.ds(j * 16, 16)]
        plsc.addupdate_scatter(acc_v, [c16], w16)
    pltpu.sync_copy(acc_v, out_hbm.at[w])


NCHUNK = E_PAD // CH


@functools.partial(
    pl.kernel,
    out_type=jax.ShapeDtypeStruct((NP, D), jnp.float32),
    mesh=_mesh,
    compiler_params=pltpu.CompilerParams(
        needs_layout_passes=False, use_tc_tiling_on_sc=False
    ),
    scratch_types=[
        pltpu.VMEM((2, CH), jnp.int32),
        pltpu.VMEM((2, CH), jnp.int32),
        pltpu.VMEM((2, CH), jnp.float32),
        pltpu.VMEM((2, CH), jnp.int32),
        pltpu.VMEM((2, CH, CPT), jnp.float32),
        pltpu.VMEM((NP, CPT), jnp.float32),
        pltpu.SemaphoreType.DMA,
        pltpu.SemaphoreType.DMA,
        pltpu.SemaphoreType.DMA,
        pltpu.SemaphoreType.DMA,
    ],
)
def _edge_kernel(yf_hbm, row_hbm, col_hbm, ew_hbm, zero_hbm, out_hbm,
                 row_v, col_v, ew_v, idx_v, gat_v, acc_v, sr0, sr1, sg0, sg1):
    w = _wid()
    srs = (sr0, sr1)
    sgs = (sg0, sg1)
    iota16 = lax.iota(jnp.int32, 16)

    def rce_copies(ci, b):
        base = ci * CH
        return (
            pltpu.make_async_copy(row_hbm.at[pl.ds(base, CH)], row_v.at[b], srs[b]),
            pltpu.make_async_copy(col_hbm.at[pl.ds(base, CH)], col_v.at[b], srs[b]),
            pltpu.make_async_copy(ew_hbm.at[pl.ds(base, CH)], ew_v.at[b], srs[b]),
        )

    def start_rce(ci, b):
        for c in rce_copies(ci, b):
            c.start()

    def wait_rce(b):
        for c in rce_copies(0, b):
            c.wait()

    def make_idx(b):
        @plsc.parallel_loop(0, CH // 16, unroll=4)
        def ibody(j):
            r16 = row_v[b, pl.ds(j * 16, 16)]
            idx_v[b, pl.ds(j * 16, 16)] = r16 * NTILES + w

    def gat_copy(b):
        return pltpu.make_async_copy(yf_hbm.at[idx_v.at[b]], gat_v.at[b], sgs[b])

    colpat = iota16 % CPT
    pairpat = iota16 // CPT

    def _vperm(vec, pat):
        return lax.gather(
            vec, pat[:, None], _DNUMS, (1,),
            mode=lax.GatherScatterMode.PROMISE_IN_BOUNDS)

    def compute(b):
        @plsc.parallel_loop(0, CH // 16, unroll=2)
        def cbody(j):
            c16 = col_v[b, pl.ds(j * 16, 16)]
            w16 = ew_v[b, pl.ds(j * 16, 16)]
            base = j * 16
            for k in range(8):
                pat = pairpat + (2 * k)
                n16 = base + pat
                data = plsc.load_gather(gat_v.at[b], [n16, colpat])
                vals = data * _vperm(w16, pat)
                dx = _vperm(c16, pat)
                plsc.addupdate_scatter(acc_v, [dx, colpat], vals)

    zc = pltpu.make_async_copy(zero_hbm, acc_v, sg0)
    zc.start()
    start_rce(0, 0)
    start_rce(1, 1)
    wait_rce(0)
    make_idx(0)
    zc.wait()
    gat_copy(0).start()

    def outer(g, _):
        for b in range(2):
            nb = 1 - b
            ci = g * 2 + b

            @pl.when(ci + 1 < NCHUNK)
            def _():
                wait_rce(nb)
                make_idx(nb)
                gat_copy(nb).start()

            gat_copy(b).wait()
            compute(b)

            @pl.when(ci + 2 < NCHUNK)
            def _():
                start_rce(ci + 2, b)

        return 0

    lax.fori_loop(0, NCHUNK // 2, outer, 0)
    pltpu.sync_copy(acc_v, out_hbm.at[:, pl.ds(w * CPT, CPT)])


def _mm_body(x_ref, w_ref, o_ref):
    o_ref[...] = jnp.dot(x_ref[...], w_ref[...], preferred_element_type=jnp.float32)


def _scale_body(p_ref, xl_ref, y_ref):
    deg = 1.0 + jnp.sum(p_ref[...], axis=0)
    dinv = jnp.where(deg > 0.0, lax.rsqrt(deg), 0.0)
    y_ref[...] = xl_ref[...] * dinv[:, None]


def _final_body(p_ref, s_ref, y_ref, b_ref, o_ref):
    deg = 1.0 + jnp.sum(p_ref[...], axis=0)
    dinv = jnp.where(deg > 0.0, lax.rsqrt(deg), 0.0)
    o_ref[...] = jnp.maximum(dinv[:, None] * (s_ref[...] + y_ref[...]) + b_ref[...], 0.0)


@jax.jit
def _impl(x, edge_index, edge_weight, W, b):
    row = edge_index[0].astype(jnp.int32)
    col = edge_index[1].astype(jnp.int32)
    pad = E_PAD - E
    rowp = jnp.concatenate([row, jnp.zeros((pad,), jnp.int32)])
    colp = jnp.concatenate([col, jnp.zeros((pad,), jnp.int32)])
    ewp = jnp.concatenate([edge_weight.astype(jnp.float32),
                           jnp.zeros((pad,), jnp.float32)])
    xp = jnp.concatenate([x.astype(jnp.float32),
                          jnp.zeros((NP - N, x.shape[1]), jnp.float32)])

    xlin = pl.pallas_call(
        _mm_body,
        grid=(NP // BN,),
        in_specs=[
            pl.BlockSpec((BN, D), lambda i: (i, 0)),
            pl.BlockSpec((D, D), lambda i: (0, 0)),
        ],
        out_specs=pl.BlockSpec((BN, D), lambda i: (i, 0)),
        out_shape=jax.ShapeDtypeStruct((NP, D), jnp.float32),
    )(xp, W)

    partials = _deg_kernel(colp, ewp)

    y = pl.pallas_call(
        _scale_body,
        grid=(NP // BN,),
        in_specs=[
            pl.BlockSpec((NTILES, BN), lambda i: (0, i)),
            pl.BlockSpec((BN, D), lambda i: (i, 0)),
        ],
        out_specs=pl.BlockSpec((BN, D), lambda i: (i, 0)),
        out_shape=jax.ShapeDtypeStruct((NP, D), jnp.float32),
    )(partials, xlin)

    yf = y.reshape(NP * NTILES, CPT)
    zero = jnp.zeros((NP, CPT), jnp.float32)
    s = _edge_kernel(yf, rowp, colp, ewp, zero)

    b2 = b.reshape(1, D).astype(jnp.float32)
    out = pl.pallas_call(
        _final_body,
        grid=(NP // BN,),
        in_specs=[
            pl.BlockSpec((NTILES, BN), lambda i: (0, i)),
            pl.BlockSpec((BN, D), lambda i: (i, 0)),
            pl.BlockSpec((BN, D), lambda i: (i, 0)),
            pl.BlockSpec((1, D), lambda i: (0, 0)),
        ],
        out_specs=pl.BlockSpec((BN, D), lambda i: (i, 0)),
        out_shape=jax.ShapeDtypeStruct((NP, D), jnp.float32),
    )(partials, s, y, b2)
    return out[:N]


def kernel(x, edge_index, edge_weight, W, b):
    return _impl(x, edge_index, edge_weight, W, b)


# speedup vs baseline: 1.4460x; 1.0177x over previous
import functools

import jax
import jax.numpy as jnp
from jax import lax
from jax.experimental import pallas as pl
from jax.experimental.pallas import tpu as pltpu
from jax.experimental.pallas import tpu_sc as plsc

N = 10000
NP = 10240
E = 160000
D = 256
NTILES = 32
CPT = D // NTILES
BN = 512
CH = 1024
E_PAD = 163840
EPW = E_PAD // NTILES

_mesh = plsc.VectorSubcoreMesh(
    core_axis_name="c", subcore_axis_name="s", num_cores=2, num_subcores=16
)

_DNUMS = lax.GatherDimensionNumbers(
    offset_dims=(), collapsed_slice_dims=(0,), start_index_map=(0,)
)


def _wid():
    return lax.axis_index("s") * 2 + lax.axis_index("c")


@functools.partial(
    pl.kernel,
    out_type=jax.ShapeDtypeStruct((NTILES, NP), jnp.float32),
    mesh=_mesh,
    compiler_params=pltpu.CompilerParams(needs_layout_passes=False),
    scratch_types=[
        pltpu.VMEM((EPW,), jnp.int32),
        pltpu.VMEM((EPW,), jnp.float32),
        pltpu.VMEM((NP,), jnp.float32),
    ],
)
def _deg_kernel(col_hbm, ew_hbm, out_hbm, col_v, ew_v, acc_v):
    w = _wid()
    base = w * EPW
    pltpu.sync_copy(col_hbm.at[pl.ds(base, EPW)], col_v)
    pltpu.sync_copy(ew_hbm.at[pl.ds(base, EPW)], ew_v)

    @plsc.parallel_loop(0, NP // 16, unroll=4)
    def zero_body(i):
        acc_v[pl.ds(i * 16, 16)] = jnp.zeros((16,), jnp.float32)

    @plsc.parallel_loop(0, EPW // 16, unroll=4)
    def ebody(j):
        c16 = col_v[pl.ds(j * 16, 16)]
        w16 = ew_v[pl.ds(j * 16, 16)]
        plsc.addupdate_scatter(acc_v, [c16], w16)
    pltpu.sync_copy(acc_v, out_hbm.at[w])


NCHUNK = E_PAD // CH


@functools.partial(
    pl.kernel,
    out_type=jax.ShapeDtypeStruct((NP, D), jnp.float32),
    mesh=_mesh,
    compiler_params=pltpu.CompilerParams(
        needs_layout_passes=False, use_tc_tiling_on_sc=False
    ),
    scratch_types=[
        pltpu.VMEM((2, CH), jnp.int32),
        pltpu.VMEM((2, CH), jnp.int32),
        pltpu.VMEM((2, CH), jnp.float32),
        pltpu.VMEM((2, CH), jnp.int32),
        pltpu.VMEM((2, CH, CPT), jnp.float32),
        pltpu.VMEM((NP, CPT), jnp.float32),
        pltpu.SemaphoreType.DMA,
        pltpu.SemaphoreType.DMA,
        pltpu.SemaphoreType.DMA,
        pltpu.SemaphoreType.DMA,
    ],
)
def _edge_kernel(yf_hbm, row_hbm, col_hbm, ew_hbm, zero_hbm, out_hbm,
                 row_v, col_v, ew_v, idx_v, gat_v, acc_v, sr0, sr1, sg0, sg1):
    w = _wid()
    srs = (sr0, sr1)
    sgs = (sg0, sg1)
    iota16 = lax.iota(jnp.int32, 16)

    def rce_copies(ci, b):
        base = ci * CH
        return (
            pltpu.make_async_copy(row_hbm.at[pl.ds(base, CH)], row_v.at[b], srs[b]),
            pltpu.make_async_copy(col_hbm.at[pl.ds(base, CH)], col_v.at[b], srs[b]),
            pltpu.make_async_copy(ew_hbm.at[pl.ds(base, CH)], ew_v.at[b], srs[b]),
        )

    def start_rce(ci, b):
        for c in rce_copies(ci, b):
            c.start()

    def wait_rce(b):
        for c in rce_copies(0, b):
            c.wait()

    def make_idx(b):
        @plsc.parallel_loop(0, CH // 16, unroll=4)
        def ibody(j):
            r16 = row_v[b, pl.ds(j * 16, 16)]
            idx_v[b, pl.ds(j * 16, 16)] = r16 * NTILES + w

    def gat_copy(b):
        return pltpu.make_async_copy(yf_hbm.at[idx_v.at[b]], gat_v.at[b], sgs[b])

    colpat = iota16 % CPT
    pairpat = iota16 // CPT

    def _vperm(vec, pat):
        return lax.gather(
            vec, pat[:, None], _DNUMS, (1,),
            mode=lax.GatherScatterMode.PROMISE_IN_BOUNDS)

    def compute(b):
        @plsc.parallel_loop(0, CH // 16, unroll=1)
        def cbody(j):
            c16 = col_v[b, pl.ds(j * 16, 16)]
            w16 = ew_v[b, pl.ds(j * 16, 16)]
            for k in range(8):
                pat = pairpat + (2 * k)
                data = plsc.load_gather(gat_v.at[b], [j * 16 + pat, colpat])
                vals = data * _vperm(w16, pat)
                dx = _vperm(c16, pat)
                plsc.addupdate_scatter(acc_v, [dx, colpat], vals)

    zc = pltpu.make_async_copy(zero_hbm, acc_v, sg0)
    zc.start()
    start_rce(0, 0)
    start_rce(1, 1)
    wait_rce(0)
    make_idx(0)
    zc.wait()
    gat_copy(0).start()

    def outer(g, _):
        for b in range(2):
            nb = 1 - b
            ci = g * 2 + b

            @pl.when(ci + 1 < NCHUNK)
            def _():
                wait_rce(nb)
                make_idx(nb)
                gat_copy(nb).start()

            gat_copy(b).wait()
            compute(b)

            @pl.when(ci + 2 < NCHUNK)
            def _():
                start_rce(ci + 2, b)

        return 0

    lax.fori_loop(0, NCHUNK // 2, outer, 0)
    pltpu.sync_copy(acc_v, out_hbm.at[:, pl.ds(w * CPT, CPT)])


def _mm_body(x_ref, w_ref, o_ref):
    o_ref[...] = jnp.dot(x_ref[...], w_ref[...], preferred_element_type=jnp.float32)


def _scale_body(p_ref, xl_ref, y_ref):
    deg = 1.0 + jnp.sum(p_ref[...], axis=0)
    dinv = jnp.where(deg > 0.0, lax.rsqrt(deg), 0.0)
    y_ref[...] = xl_ref[...] * dinv[:, None]


def _final_body(p_ref, s_ref, y_ref, b_ref, o_ref):
    deg = 1.0 + jnp.sum(p_ref[...], axis=0)
    dinv = jnp.where(deg > 0.0, lax.rsqrt(deg), 0.0)
    o_ref[...] = jnp.maximum(dinv[:, None] * (s_ref[...] + y_ref[...]) + b_ref[...], 0.0)


@jax.jit
def _impl(x, edge_index, edge_weight, W, b):
    row = edge_index[0].astype(jnp.int32)
    col = edge_index[1].astype(jnp.int32)
    pad = E_PAD - E
    rowp = jnp.concatenate([row, jnp.zeros((pad,), jnp.int32)])
    colp = jnp.concatenate([col, jnp.zeros((pad,), jnp.int32)])
    ewp = jnp.concatenate([edge_weight.astype(jnp.float32),
                           jnp.zeros((pad,), jnp.float32)])
    xp = jnp.concatenate([x.astype(jnp.float32),
                          jnp.zeros((NP - N, x.shape[1]), jnp.float32)])

    xlin = pl.pallas_call(
        _mm_body,
        grid=(NP // BN,),
        in_specs=[
            pl.BlockSpec((BN, D), lambda i: (i, 0)),
            pl.BlockSpec((D, D), lambda i: (0, 0)),
        ],
        out_specs=pl.BlockSpec((BN, D), lambda i: (i, 0)),
        out_shape=jax.ShapeDtypeStruct((NP, D), jnp.float32),
    )(xp, W)

    partials = _deg_kernel(colp, ewp)

    y = pl.pallas_call(
        _scale_body,
        grid=(NP // BN,),
        in_specs=[
            pl.BlockSpec((NTILES, BN), lambda i: (0, i)),
            pl.BlockSpec((BN, D), lambda i: (i, 0)),
        ],
        out_specs=pl.BlockSpec((BN, D), lambda i: (i, 0)),
        out_shape=jax.ShapeDtypeStruct((NP, D), jnp.float32),
    )(partials, xlin)

    yf = y.reshape(NP * NTILES, CPT)
    zero = jnp.zeros((NP, CPT), jnp.float32)
    s = _edge_kernel(yf, rowp, colp, ewp, zero)

    b2 = b.reshape(1, D).astype(jnp.float32)
    out = pl.pallas_call(
        _final_body,
        grid=(NP // BN,),
        in_specs=[
            pl.BlockSpec((NTILES, BN), lambda i: (0, i)),
            pl.BlockSpec((BN, D), lambda i: (i, 0)),
            pl.BlockSpec((BN, D), lambda i: (i, 0)),
            pl.BlockSpec((1, D), lambda i: (0, 0)),
        ],
        out_specs=pl.BlockSpec((BN, D), lambda i: (i, 0)),
        out_shape=jax.ShapeDtypeStruct((NP, D), jnp.float32),
    )(partials, s, y, b2)
    return out[:N]


def kernel(x, edge_index, edge_weight, W, b):
    return _impl(x, edge_index, edge_weight, W, b)
